# Initial kernel scaffold; baseline (speedup 1.0000x reference)
import jax
import jax.numpy as jnp
from jax.experimental import pallas as pl


def kernel(drawings):
    raise NotImplementedError("write your pallas kernel here")


# speedup vs baseline: 19.5422x; 19.5422x over previous
import functools

import jax
import jax.numpy as jnp
from jax import lax
from jax.experimental import pallas as pl
from jax.experimental.pallas import tpu as pltpu
from jax.experimental.pallas import tpu_sc as plsc

H = 512
W = 512
N = 128
S = 32
T = 256
INK_PP = 2.0
INK_MAX_DIST = 2.0

NC = 2
NS = 16
HW = H * W
SLICE = HW // NS
D_PER_C = N // NC
S_PER_T = S // NS
CHUNKS = T // 16


def _rsqrt(d2):
    bits = lax.bitcast_convert_type(d2, jnp.int32)
    y = lax.bitcast_convert_type(
        jnp.int32(0x5F3759DF) - lax.shift_right_logical(bits, 1), jnp.float32)
    for _ in range(3):
        y = y * (1.5 - 0.5 * d2 * y * y)
    return y


def _body(xs_hbm, ys_hbm, out_hbm, canvas, xbuf, ybuf, dbuf, idxb, valb, zbuf):
    c = lax.axis_index("c")
    s = lax.axis_index("s")
    slice_base = s * SLICE
    lane = lax.iota(jnp.int32, 16)

    def _zinit(i, _):
        zbuf[pl.ds(i * 16, 16)] = jnp.zeros((16,), jnp.float32)
        return 0
    lax.fori_loop(0, SLICE // 16, _zinit, 0)
    xbuf[pl.ds(T, 16)] = jnp.zeros((16,), jnp.float32)
    ybuf[pl.ds(T, 16)] = jnp.zeros((16,), jnp.float32)
    pltpu.sync_copy(zbuf, canvas.at[pl.ds(slice_base, SLICE)])

    def draw_body(k, _):
        d = c * D_PER_C + k
        for s2 in range(S_PER_T):
            st = s * S_PER_T + s2
            pltpu.sync_copy(xs_hbm.at[d, st], xbuf.at[pl.ds(0, T)])
            pltpu.sync_copy(ys_hbm.at[d, st], ybuf.at[pl.ds(0, T)])

            def pass1(ci, acc):
                b = ci * 16
                xa = xbuf[pl.ds(b, 16)]
                xn = xbuf[pl.ds(b + 1, 16)]
                ya = ybuf[pl.ds(b, 16)]
                yn = ybuf[pl.ds(b + 1, 16)]
                dx = xn - xa
                dy = yn - ya
                d2 = jnp.maximum(dx * dx + dy * dy, 1e-24)
                dist = jnp.minimum(d2 * _rsqrt(d2), INK_MAX_DIST)
                pad = jnp.logical_and(lane == 15, ci == CHUNKS - 1)
                dist = jnp.where(pad, 0.0, dist)
                dbuf[pl.ds(b + 1, 16)] = dist
                return acc + dist

            acc = lax.fori_loop(0, CHUNKS, pass1,
                                jnp.zeros((16,), jnp.float32))
            v0 = dbuf[pl.ds(0, 16)]
            d0 = dbuf[pl.ds(1, 16)][0]
            dbuf[pl.ds(0, 16)] = jnp.where(lane == 0, d0, v0)
            sumink = jnp.sum(acc) + d0

            sv = jnp.full((16,), sumink, jnp.float32)
            tiny = sv < 2.22e-06
            small = sv < INK_PP
            a = jnp.where(tiny, 0.0,
                          jnp.where(small,
                                    INK_PP / jnp.maximum(sv, 1e-20),
                                    INK_PP / INK_MAX_DIST))
            b0 = jnp.where(tiny, INK_PP / T, 0.0)

            def pass2(ci, _):
                b = ci * 16
                x = 0.0 - xbuf[pl.ds(b, 16)]
                y = ybuf[pl.ds(b, 16)]
                ink = a * dbuf[pl.ds(b, 16)] + b0
                ixf = x.astype(jnp.int32)
                iyf = y.astype(jnp.int32)
                fx = x - ixf.astype(jnp.float32)
                fy = y - iyf.astype(jnp.float32)
                gx = 1.0 - fx
                gy = 1.0 - fy
                base_i = ixf * W + iyf
                row = s2 * 8 + lax.div(ci, 2)
                col = lax.rem(ci, 2) * 64
                idxb[row, pl.ds(col, 16)] = base_i
                valb[row, pl.ds(col, 16)] = ink * gx * gy
                idxb[row, pl.ds(col + 16, 16)] = base_i + W
                valb[row, pl.ds(col + 16, 16)] = ink * fx * gy
                idxb[row, pl.ds(col + 32, 16)] = base_i + 1
                valb[row, pl.ds(col + 32, 16)] = ink * gx * fy
                idxb[row, pl.ds(col + 48, 16)] = base_i + W + 1
                valb[row, pl.ds(col + 48, 16)] = ink * fx * fy
                return 0

            lax.fori_loop(0, CHUNKS, pass2, 0)

        plsc.subcore_barrier()
        for j in range(16):
            pltpu.sync_copy(valb.at[j], canvas.at[idxb.at[j]], add=True)
        plsc.subcore_barrier()
        pltpu.sync_copy(canvas.at[pl.ds(slice_base, SLICE)],
                        out_hbm.at[d, pl.ds(slice_base, SLICE)])
        pltpu.sync_copy(zbuf, canvas.at[pl.ds(slice_base, SLICE)])
        return 0

    lax.fori_loop(0, D_PER_C, draw_body, 0)


@jax.jit
def kernel(drawings):
    xs = drawings[..., 1]
    ys = drawings[..., 0]
    mesh = plsc.VectorSubcoreMesh(core_axis_name="c", subcore_axis_name="s",
                                  num_cores=NC, num_subcores=NS)
    paint = pl.kernel(
        _body,
        out_type=jax.ShapeDtypeStruct((N, HW), jnp.float32),
        mesh=mesh,
        compiler_params=pltpu.CompilerParams(needs_layout_passes=False),
        scratch_types=[
            pltpu.VMEM_SHARED((HW,), jnp.float32),
            pltpu.VMEM((T + 16,), jnp.float32),
            pltpu.VMEM((T + 16,), jnp.float32),
            pltpu.VMEM((T + 16,), jnp.float32),
            pltpu.VMEM((16, 128), jnp.int32),
            pltpu.VMEM((16, 128), jnp.float32),
            pltpu.VMEM((SLICE,), jnp.float32),
        ],
    )
    return paint(xs, ys).reshape(N, H, W)


# speedup vs baseline: 24.0139x; 1.2288x over previous
import functools

import jax
import jax.numpy as jnp
from jax import lax
from jax.experimental import pallas as pl
from jax.experimental.pallas import tpu as pltpu
from jax.experimental.pallas import tpu_sc as plsc

H = 512
W = 512
N = 128
S = 32
T = 256
INK_PP = 2.0
INK_MAX_DIST = 2.0

NC = 2
NS = 16
HW = H * W
SLICE = HW // NS
D_PER_C = N // NC
S_PER_T = S // NS
CHUNKS = T // 16
NCAN = 2
NSTG = 2


def _rsqrt(d2):
    bits = lax.bitcast_convert_type(d2, jnp.int32)
    y = lax.bitcast_convert_type(
        jnp.int32(0x5F3759DF) - lax.shift_right_logical(bits, 1), jnp.float32)
    for _ in range(3):
        y = y * (1.5 - 0.5 * d2 * y * y)
    return y


def _body(xs_hbm, ys_hbm, out_hbm, canvas, xbuf, ybuf, dbuf, idxb, valb, zbuf,
          scat_sem, wb_sem, z_sem):
    c = lax.axis_index("c")
    s = lax.axis_index("s")
    slice_base = s * SLICE
    lane = lax.iota(jnp.int32, 16)

    def _zinit(i, _):
        zbuf[pl.ds(i * 16, 16)] = jnp.zeros((16,), jnp.float32)
        return 0
    lax.fori_loop(0, SLICE // 16, _zinit, 0)
    for s2 in range(S_PER_T):
        xbuf[s2, pl.ds(T, 16)] = jnp.zeros((16,), jnp.float32)
        ybuf[s2, pl.ds(T, 16)] = jnp.zeros((16,), jnp.float32)
    for m in range(NCAN):
        pltpu.sync_copy(zbuf, canvas.at[pl.ds(m * HW + slice_base, SLICE)])

    def fire_scat(ws):
        for j in range(16):
            pltpu.async_copy(valb.at[ws, j], canvas.at[idxb.at[ws, j]],
                             scat_sem, add=True)

    def drain_scat(ws):
        for j in range(16):
            pltpu.make_async_copy(valb.at[ws, j],
                                  canvas.at[idxb.at[ws, j]], scat_sem).wait()

    def _wb_copy(kk, cs):
        d = c * D_PER_C + kk
        return (canvas.at[pl.ds(cs * HW + slice_base, SLICE)],
                out_hbm.at[d, pl.ds(slice_base, SLICE)])

    def fire_wb(kk, cs):
        src, dst = _wb_copy(kk, cs)
        pltpu.async_copy(src, dst, wb_sem)

    def wait_wb(kk, cs):
        src, dst = _wb_copy(kk, cs)
        pltpu.make_async_copy(src, dst, wb_sem).wait()

    def _z_copy(cs):
        return (zbuf, canvas.at[pl.ds(cs * HW + slice_base, SLICE)])

    def fire_zero(cs):
        src, dst = _z_copy(cs)
        pltpu.async_copy(src, dst, z_sem)

    def wait_zero(cs):
        src, dst = _z_copy(cs)
        pltpu.make_async_copy(src, dst, z_sem).wait()

    def compute(kk):
        d = c * D_PER_C + kk
        for s2 in range(S_PER_T):
            st = s * S_PER_T + s2
            pltpu.sync_copy(xs_hbm.at[d, st], xbuf.at[s2, pl.ds(0, T)])
            pltpu.sync_copy(ys_hbm.at[d, st], ybuf.at[s2, pl.ds(0, T)])
        ws = lax.rem(kk, NSTG)
        cbase = lax.rem(kk, NCAN) * HW
        for s2 in range(S_PER_T):
            def pass1(ci, acc):
                b = ci * 16
                xa = xbuf[s2, pl.ds(b, 16)]
                xn = xbuf[s2, pl.ds(b + 1, 16)]
                ya = ybuf[s2, pl.ds(b, 16)]
                yn = ybuf[s2, pl.ds(b + 1, 16)]
                dx = xn - xa
                dy = yn - ya
                d2 = jnp.maximum(dx * dx + dy * dy, 1e-24)
                dist = jnp.minimum(d2 * _rsqrt(d2), INK_MAX_DIST)
                pad = jnp.logical_and(lane == 15, ci == CHUNKS - 1)
                dist = jnp.where(pad, 0.0, dist)
                dbuf[s2, pl.ds(b + 1, 16)] = dist
                return acc + dist

            acc = lax.fori_loop(0, CHUNKS, pass1,
                                jnp.zeros((16,), jnp.float32))
            v0 = dbuf[s2, pl.ds(0, 16)]
            d0 = dbuf[s2, pl.ds(1, 16)][0]
            dbuf[s2, pl.ds(0, 16)] = jnp.where(lane == 0, d0, v0)
            sumink = jnp.sum(acc) + d0

            sv = jnp.full((16,), sumink, jnp.float32)
            tiny = sv < 2.22e-06
            small = sv < INK_PP
            a = jnp.where(tiny, 0.0,
                          jnp.where(small,
                                    INK_PP / jnp.maximum(sv, 1e-20),
                                    INK_PP / INK_MAX_DIST))
            b0 = jnp.where(tiny, INK_PP / T, 0.0)

            def make_pass2(u):
                def pass2(ci, _):
                    b = ci * 16
                    x = 0.0 - xbuf[s2, pl.ds(b, 16)]
                    y = ybuf[s2, pl.ds(b, 16)]
                    ink = a * dbuf[s2, pl.ds(b, 16)] + b0
                    ixf = x.astype(jnp.int32)
                    iyf = y.astype(jnp.int32)
                    fx = x - ixf.astype(jnp.float32)
                    fy = y - iyf.astype(jnp.float32)
                    gx = 1.0 - fx
                    gy = 1.0 - fy
                    base_i = ixf * W + iyf + cbase
                    row = s2 * 8 + lax.div(ci, 2)
                    col = lax.rem(ci, 2) * 64
                    idxb[u, row, pl.ds(col, 16)] = base_i
                    valb[u, row, pl.ds(col, 16)] = ink * gx * gy
                    idxb[u, row, pl.ds(col + 16, 16)] = base_i + W
                    valb[u, row, pl.ds(col + 16, 16)] = ink * fx * gy
                    idxb[u, row, pl.ds(col + 32, 16)] = base_i + 1
                    valb[u, row, pl.ds(col + 32, 16)] = ink * gx * fy
                    idxb[u, row, pl.ds(col + 48, 16)] = base_i + W + 1
                    valb[u, row, pl.ds(col + 48, 16)] = ink * fx * fy
                    return 0
                return pass2

            for u in range(NSTG):
                @pl.when(ws == u)
                def _(u=u):
                    lax.fori_loop(0, CHUNKS, make_pass2(u), 0)

    compute(0)

    def _dispatch(kmod, fn):
        for u in range(NSTG):
            @pl.when(kmod == u)
            def _(u=u):
                fn(u)

    def loop(k, _):
        @pl.when(k >= 2)
        def _():
            wait_zero(lax.rem(k, NCAN))
        @pl.when(k >= 1)
        def _():
            wait_wb(k - 1, lax.rem(k + 1, NCAN))
        @pl.when(jnp.logical_and(k >= 1, k <= D_PER_C - 2))
        def _():
            fire_zero(lax.rem(k + 1, NCAN))
        plsc.subcore_barrier()
        _dispatch(lax.rem(k, NSTG), fire_scat)
        @pl.when(k < D_PER_C - 1)
        def _():
            compute(k + 1)
        _dispatch(lax.rem(k, NSTG), drain_scat)
        plsc.subcore_barrier()
        fire_wb(k, lax.rem(k, NCAN))
        return 0

    lax.fori_loop(0, D_PER_C, loop, 0)

    wait_wb(D_PER_C - 1, (D_PER_C - 1) % NCAN)


@jax.jit
def kernel(drawings):
    xs = drawings[..., 1]
    ys = drawings[..., 0]
    mesh = plsc.VectorSubcoreMesh(core_axis_name="c", subcore_axis_name="s",
                                  num_cores=NC, num_subcores=NS)
    paint = pl.kernel(
        _body,
        out_type=jax.ShapeDtypeStruct((N, HW), jnp.float32),
        mesh=mesh,
        compiler_params=pltpu.CompilerParams(needs_layout_passes=False),
        scratch_types=[
            pltpu.VMEM_SHARED((NCAN * HW,), jnp.float32),
            pltpu.VMEM((S_PER_T, T + 16), jnp.float32),
            pltpu.VMEM((S_PER_T, T + 16), jnp.float32),
            pltpu.VMEM((S_PER_T, T + 16), jnp.float32),
            pltpu.VMEM((NSTG, 16, 128), jnp.int32),
            pltpu.VMEM((NSTG, 16, 128), jnp.float32),
            pltpu.VMEM((SLICE,), jnp.float32),
            pltpu.SemaphoreType.DMA,
            pltpu.SemaphoreType.DMA,
            pltpu.SemaphoreType.DMA,
        ],
    )
    return paint(xs, ys).reshape(N, H, W)
